# baseline (speedup 1.0000x reference)
import jax
import jax.numpy as jnp
from jax.experimental import pallas as pl
from jax.experimental.pallas import tpu as pltpu

N_TOKENS = 16384
IN_DIM = 1024
HIDDEN = 256
NUM_EXPERTS = 16

CHUNK = 1024
NCHUNK = N_TOKENS // CHUNK
NBUF = 4


def _dot(a, b):
    return jax.lax.dot_general(
        a, b, (((1,), (0,)), ((), ())), preferred_element_type=jnp.float32
    )


def _route_tile(x, w1, b1, w2, b2, lt, iota_row):
    h = _dot(x.astype(jnp.bfloat16), w1)
    h = jnp.maximum(h + b1, 0.0)
    logits = _dot(h.astype(jnp.bfloat16), w2)
    logits = logits + b2
    m = jnp.max(logits, axis=-1, keepdims=True)
    e = jnp.exp(logits - m)
    probs = e / jnp.sum(e, axis=-1, keepdims=True)
    probs_t = jax.lax.transpose(probs, (1, 0))

    mask = (logits == m).astype(jnp.bfloat16)
    cnt = _dot(mask, lt)
    first = jnp.where(cnt == 1.0, mask.astype(jnp.float32), 0.0).astype(jnp.bfloat16)
    first_t = jax.lax.transpose(first, (1, 0))
    eid_lane = _dot(iota_row, first_t)
    return probs_t, eid_lane.astype(jnp.int32).reshape(CHUNK)


def _router_body(x_hbm, w1_ref, b1_ref, w2_ref, b2_ref, p_ref, e_ref,
                 xbuf, sems):
    w1 = w1_ref[...].astype(jnp.bfloat16)
    b1 = b1_ref[...].reshape(1, HIDDEN)
    w2 = w2_ref[...].astype(jnp.bfloat16)
    b2 = b2_ref[...].reshape(1, NUM_EXPERTS)
    lt = (
        jax.lax.broadcasted_iota(jnp.int32, (NUM_EXPERTS, NUM_EXPERTS), 0)
        <= jax.lax.broadcasted_iota(jnp.int32, (NUM_EXPERTS, NUM_EXPERTS), 1)
    ).astype(jnp.bfloat16)
    iota_row = jax.lax.broadcasted_iota(
        jnp.int32, (1, NUM_EXPERTS), 1
    ).astype(jnp.bfloat16)

    def chunk_copy(i, buf):
        return pltpu.make_async_copy(
            x_hbm.at[pl.ds(i * CHUNK, CHUNK), :],
            xbuf.at[buf],
            sems.at[buf],
        )

    for b in range(NBUF):
        chunk_copy(b, b).start()

    def loop(i, carry):
        buf = jax.lax.rem(i, NBUF)
        chunk_copy(i, buf).wait()
        pt, el = _route_tile(xbuf[buf], w1, b1, w2, b2, lt, iota_row)
        p_ref[:, pl.ds(i * CHUNK, CHUNK)] = pt
        e_ref[pl.ds(i * CHUNK, CHUNK)] = el
        nxt = i + NBUF

        @pl.when(nxt < NCHUNK)
        def _():
            chunk_copy(nxt, buf).start()

        return carry

    jax.lax.fori_loop(0, NCHUNK, loop, 0)


def kernel(patch_feat, W1, b1, W2, b2):
    probs_t, eid = pl.pallas_call(
        _router_body,
        in_specs=[
            pl.BlockSpec(memory_space=pltpu.MemorySpace.HBM),
            pl.BlockSpec((IN_DIM, HIDDEN), lambda: (0, 0)),
            pl.BlockSpec((HIDDEN,), lambda: (0,)),
            pl.BlockSpec((HIDDEN, NUM_EXPERTS), lambda: (0, 0)),
            pl.BlockSpec((NUM_EXPERTS,), lambda: (0,)),
        ],
        out_specs=[
            pl.BlockSpec((NUM_EXPERTS, N_TOKENS), lambda: (0, 0)),
            pl.BlockSpec((N_TOKENS,), lambda: (0,)),
        ],
        out_shape=[
            jax.ShapeDtypeStruct((NUM_EXPERTS, N_TOKENS), jnp.float32),
            jax.ShapeDtypeStruct((N_TOKENS,), jnp.int32),
        ],
        scratch_shapes=[
            pltpu.VMEM((NBUF, CHUNK, IN_DIM), jnp.float32),
            pltpu.SemaphoreType.DMA((NBUF,)),
        ],
    )(patch_feat, W1, b1, W2, b2)
    return probs_t.T, eid


# speedup vs baseline: 1.1051x; 1.1051x over previous
import jax
import jax.numpy as jnp
from jax.experimental import pallas as pl
from jax.experimental.pallas import tpu as pltpu

N_TOKENS = 16384
IN_DIM = 1024
HIDDEN = 256
NUM_EXPERTS = 16

CHUNK = 2048
NCHUNK = N_TOKENS // CHUNK
NBUF = 3


def _dot(a, b):
    return jax.lax.dot_general(
        a, b, (((1,), (0,)), ((), ())), preferred_element_type=jnp.float32
    )


def _route_tile(x, w1, b1, w2, b2, lt, iota_row):
    h = _dot(x.astype(jnp.bfloat16), w1)
    h = jnp.maximum(h + b1, 0.0)
    logits = _dot(h.astype(jnp.bfloat16), w2)
    logits = logits + b2
    m = jnp.max(logits, axis=-1, keepdims=True)
    e = jnp.exp(logits - m)
    probs = e / jnp.sum(e, axis=-1, keepdims=True)
    probs_t = jax.lax.transpose(probs, (1, 0))

    mask = (logits == m).astype(jnp.bfloat16)
    cnt = _dot(mask, lt)
    first = jnp.where(cnt == 1.0, mask.astype(jnp.float32), 0.0).astype(jnp.bfloat16)
    first_t = jax.lax.transpose(first, (1, 0))
    eid_lane = _dot(iota_row, first_t)
    return probs_t, eid_lane.astype(jnp.int32).reshape(CHUNK)


def _router_body(x_hbm, w1_ref, b1_ref, w2_ref, b2_ref, p_ref, e_ref,
                 xbuf, sems):
    w1 = w1_ref[...].astype(jnp.bfloat16)
    b1 = b1_ref[...].reshape(1, HIDDEN)
    w2 = w2_ref[...].astype(jnp.bfloat16)
    b2 = b2_ref[...].reshape(1, NUM_EXPERTS)
    lt = (
        jax.lax.broadcasted_iota(jnp.int32, (NUM_EXPERTS, NUM_EXPERTS), 0)
        <= jax.lax.broadcasted_iota(jnp.int32, (NUM_EXPERTS, NUM_EXPERTS), 1)
    ).astype(jnp.bfloat16)
    iota_row = jax.lax.broadcasted_iota(
        jnp.int32, (1, NUM_EXPERTS), 1
    ).astype(jnp.bfloat16)

    def chunk_copy(i, buf):
        return pltpu.make_async_copy(
            x_hbm.at[pl.ds(i * CHUNK, CHUNK), :],
            xbuf.at[buf],
            sems.at[buf],
        )

    for b in range(NBUF):
        chunk_copy(b, b).start()

    def loop(i, carry):
        buf = jax.lax.rem(i, NBUF)
        chunk_copy(i, buf).wait()
        pt, el = _route_tile(xbuf[buf], w1, b1, w2, b2, lt, iota_row)
        p_ref[:, pl.ds(i * CHUNK, CHUNK)] = pt
        e_ref[pl.ds(i * CHUNK, CHUNK)] = el
        nxt = i + NBUF

        @pl.when(nxt < NCHUNK)
        def _():
            chunk_copy(nxt, buf).start()

        return carry

    jax.lax.fori_loop(0, NCHUNK, loop, 0)


def kernel(patch_feat, W1, b1, W2, b2):
    probs_t, eid = pl.pallas_call(
        _router_body,
        in_specs=[
            pl.BlockSpec(memory_space=pltpu.MemorySpace.HBM),
            pl.BlockSpec((IN_DIM, HIDDEN), lambda: (0, 0)),
            pl.BlockSpec((HIDDEN,), lambda: (0,)),
            pl.BlockSpec((HIDDEN, NUM_EXPERTS), lambda: (0, 0)),
            pl.BlockSpec((NUM_EXPERTS,), lambda: (0,)),
        ],
        out_specs=[
            pl.BlockSpec((NUM_EXPERTS, N_TOKENS), lambda: (0, 0)),
            pl.BlockSpec((N_TOKENS,), lambda: (0,)),
        ],
        out_shape=[
            jax.ShapeDtypeStruct((NUM_EXPERTS, N_TOKENS), jnp.float32),
            jax.ShapeDtypeStruct((N_TOKENS,), jnp.int32),
        ],
        scratch_shapes=[
            pltpu.VMEM((NBUF, CHUNK, IN_DIM), jnp.float32),
            pltpu.SemaphoreType.DMA((NBUF,)),
        ],
    )(patch_feat, W1, b1, W2, b2)
    return probs_t.T, eid
